# Initial kernel scaffold; baseline (speedup 1.0000x reference)
import jax
import jax.numpy as jnp
from jax.experimental import pallas as pl


def kernel(x, edge_index, edge_attr, batch, embed, conv0_W1, conv0_W2, conv1_W1, conv1_W2, lin0_W, lin0_b, lin1_W, lin1_b, wprop_W, wprop_b):
    raise NotImplementedError("write your pallas kernel here")


# speedup vs baseline: 3.2041x; 3.2041x over previous
import functools

import numpy as np
import jax
import jax.numpy as jnp
from jax import lax
from jax.experimental import pallas as pl
from jax.experimental.pallas import tpu as pltpu
from jax.experimental.pallas import tpu_sc as plsc

N = 10000
E = 320000
H = 16
NUM_BASIS = 10
MAX_RADIUS = 2.0
NUM_GRAPHS = 256
IN_CHANNELS = 100
FC_HIDDEN = 256

NUM_WORKERS = 32
EDGES_PER_WORKER = E // NUM_WORKERS
CHUNK = 2000
NCHUNKS = EDGES_PER_WORKER // CHUNK

BE = 2000
NB = N // 1000


def _sc_mesh():
    return plsc.VectorSubcoreMesh(core_axis_name="c", subcore_axis_name="s")


@functools.lru_cache(maxsize=None)
def _make_gather(num_tables):
    out_type = [jax.ShapeDtypeStruct((E, H), jnp.float32)
                for _ in range(num_tables)]

    @functools.partial(
        pl.kernel,
        mesh=_sc_mesh(),
        out_type=out_type,
        compiler_params=pltpu.CompilerParams(use_tc_tiling_on_sc=False),
        scratch_types=[
            pltpu.VMEM((CHUNK,), jnp.int32),
            pltpu.VMEM((CHUNK, H), jnp.float32),
            pltpu.SemaphoreType.DMA,
        ],
    )
    def gather_kernel(*refs):
        tables = refs[:num_tables]
        idx_hbm = refs[num_tables]
        outs = refs[num_tables + 1:num_tables + 1 + num_tables]
        idx_v, rows_v, sem = refs[num_tables + 1 + num_tables:]
        cid = lax.axis_index("c")
        sid = lax.axis_index("s")
        wid = cid * (NUM_WORKERS // 2) + sid
        base = wid * EDGES_PER_WORKER
        for c in range(NCHUNKS):
            off = base + c * CHUNK
            pltpu.sync_copy(idx_hbm.at[pl.ds(off, CHUNK)], idx_v)
            for t in range(num_tables):
                pltpu.async_copy(tables[t].at[idx_v], rows_v, sem).wait()
                pltpu.sync_copy(rows_v, outs[t].at[pl.ds(off, CHUNK)])

    return gather_kernel


@functools.lru_cache(maxsize=None)
def _make_scatter():
    @functools.partial(
        pl.kernel,
        mesh=_sc_mesh(),
        out_type=[jax.ShapeDtypeStruct((N, H), jnp.float32),
                  jax.ShapeDtypeStruct((N, H), jnp.float32)],
        compiler_params=pltpu.CompilerParams(use_tc_tiling_on_sc=False),
        scratch_types=[
            pltpu.VMEM((CHUNK,), jnp.int32),
            pltpu.VMEM((CHUNK, H), jnp.float32),
            pltpu.VMEM_SHARED((N, H), jnp.float32),
            pltpu.SemaphoreType.DMA,
        ],
    )
    def scatter_kernel(ef_hbm, dst_hbm, zeros_hbm, out_a, out_b,
                       idx_v, rows_v, acc, sem):
        cid = lax.axis_index("c")
        sid = lax.axis_index("s")
        wid = cid * (NUM_WORKERS // 2) + sid

        @pl.when(sid == 0)
        def _():
            pltpu.sync_copy(zeros_hbm, acc)

        plsc.subcore_barrier()

        base = wid * EDGES_PER_WORKER
        for c in range(NCHUNKS):
            off = base + c * CHUNK
            pltpu.sync_copy(dst_hbm.at[pl.ds(off, CHUNK)], idx_v)
            pltpu.sync_copy(ef_hbm.at[pl.ds(off, CHUNK)], rows_v)
            pltpu.sync_copy(rows_v, acc.at[idx_v], add=True)

        plsc.subcore_barrier()

        rows_out = N // 10

        @pl.when(sid < 10)
        def _():
            r0 = sid * rows_out

            @pl.when(cid == 0)
            def _():
                pltpu.sync_copy(acc.at[pl.ds(r0, rows_out)],
                                out_a.at[pl.ds(r0, rows_out)])

            @pl.when(cid == 1)
            def _():
                pltpu.sync_copy(acc.at[pl.ds(r0, rows_out)],
                                out_b.at[pl.ds(r0, rows_out)])

    return scatter_kernel


def _edge_body(num_x, *refs):
    ea_ref = refs[0]
    xs = refs[1:1 + num_x]
    w1_ref, w2_ref, rep_ref, sel_ref, out_ref = refs[1 + num_x:]

    ea = ea_ref[...]
    r = jnp.sqrt(jnp.sum(ea * ea, axis=1, keepdims=True))
    step = MAX_RADIUS / (NUM_BASIS + 1)
    kk = (lax.broadcasted_iota(jnp.int32, (1, NUM_BASIS), 1)
          + 1).astype(jnp.float32)
    diff = (r - kk * step) / step

    def sus(t):
        ts = jnp.where(t > 0.0, t, 1.0)
        return jnp.where(t > 0.0, jnp.exp(-1.0 / ts), 0.0)

    emb = sus(diff + 1.0) * sus(1.0 - diff)
    h1 = jnp.dot(emb, w1_ref[...], preferred_element_type=jnp.float32)
    h1 = jnp.maximum(h1, 0.0)
    w = jnp.dot(h1, w2_ref[...], preferred_element_type=jnp.float32)

    x1 = xs[0][...]
    for t in range(1, num_x):
        x1 = x1 + xs[t][...]
    xr = jnp.dot(x1, rep_ref[...], preferred_element_type=jnp.float32)
    out_ref[...] = jnp.dot(w * xr, sel_ref[...],
                           preferred_element_type=jnp.float32)


def _make_edge_call(num_x):
    grid = (E // BE,)
    in_specs = (
        [pl.BlockSpec((BE, 3), lambda i: (i, 0))]
        + [pl.BlockSpec((BE, H), lambda i: (i, 0)) for _ in range(num_x)]
        + [
            pl.BlockSpec((NUM_BASIS, FC_HIDDEN), lambda i: (0, 0)),
            pl.BlockSpec((FC_HIDDEN, H * H), lambda i: (0, 0)),
            pl.BlockSpec((H, H * H), lambda i: (0, 0)),
            pl.BlockSpec((H * H, H), lambda i: (0, 0)),
        ]
    )
    return pl.pallas_call(
        functools.partial(_edge_body, num_x),
        grid=grid,
        in_specs=in_specs,
        out_specs=pl.BlockSpec((BE, H), lambda i: (i, 0)),
        out_shape=jax.ShapeDtypeStruct((E, H), jnp.float32),
    )


_edge_call1 = _make_edge_call(1)
_edge_call2 = _make_edge_call(2)


def _embed_body(x_ref, table_ref, out_ref):
    xb = x_ref[...]
    classes = lax.broadcasted_iota(jnp.int32, (1, IN_CHANNELS), 1)
    onehot = (xb == classes).astype(jnp.float32)
    out_ref[...] = jnp.dot(onehot, table_ref[...],
                           preferred_element_type=jnp.float32)


_embed_call = pl.pallas_call(
    _embed_body,
    grid=(NB,),
    in_specs=[
        pl.BlockSpec((N // NB, 1), lambda i: (i, 0)),
        pl.BlockSpec((IN_CHANNELS, H), lambda i: (0, 0)),
    ],
    out_specs=pl.BlockSpec((N // NB, H), lambda i: (i, 0)),
    out_shape=jax.ShapeDtypeStruct((N, H), jnp.float32),
)


def _tail_body(pa_ref, pb_ref, b_ref, l0w_ref, l0b_ref, l1w_ref, l1b_ref,
               pw_ref, pbias_ref, out_ref, macc):
    i = pl.program_id(0)

    @pl.when(i == 0)
    def _():
        macc[...] = jnp.zeros_like(macc)

    h = pa_ref[...] + pb_ref[...]
    h = jnp.maximum(
        jnp.dot(h, l0w_ref[...], preferred_element_type=jnp.float32)
        + l0b_ref[...], 0.0)
    h = jnp.maximum(
        jnp.dot(h, l1w_ref[...], preferred_element_type=jnp.float32)
        + l1b_ref[...], 0.0)
    gids = lax.broadcasted_iota(jnp.int32, (1, NUM_GRAPHS), 1)
    onehot = (b_ref[...] == gids).astype(jnp.float32)
    macc[...] += lax.dot_general(onehot, h, (((0,), (0,)), ((), ())),
                                 preferred_element_type=jnp.float32)

    @pl.when(i == pl.num_programs(0) - 1)
    def _():
        out_ref[...] = (jnp.dot(macc[...], pw_ref[...],
                                preferred_element_type=jnp.float32)
                        + pbias_ref[...])


_tail_call = pl.pallas_call(
    _tail_body,
    grid=(NB,),
    in_specs=[
        pl.BlockSpec((N // NB, H), lambda i: (i, 0)),
        pl.BlockSpec((N // NB, H), lambda i: (i, 0)),
        pl.BlockSpec((N // NB, 1), lambda i: (i, 0)),
        pl.BlockSpec((H, H), lambda i: (0, 0)),
        pl.BlockSpec((1, H), lambda i: (0, 0)),
        pl.BlockSpec((H, H), lambda i: (0, 0)),
        pl.BlockSpec((1, H), lambda i: (0, 0)),
        pl.BlockSpec((H, 1), lambda i: (0, 0)),
        pl.BlockSpec((1, 1), lambda i: (0, 0)),
    ],
    out_specs=pl.BlockSpec((NUM_GRAPHS, 1), lambda i: (0, 0)),
    out_shape=jax.ShapeDtypeStruct((NUM_GRAPHS, 1), jnp.float32),
    scratch_shapes=[pltpu.VMEM((NUM_GRAPHS, H), jnp.float32)],
)


_REP = np.repeat(np.eye(H, dtype=np.float32), H, axis=1)
_SEL = np.tile(np.eye(H, dtype=np.float32), (H, 1))
_OUT_SCALE = 1.0 / (np.sqrt(H) * np.sqrt(E / N))
_EMB_SCALE = 1.14136 * np.exp(2.0)


def kernel(x, edge_index, edge_attr, batch, embed,
           conv0_W1, conv0_W2, conv1_W1, conv1_W2,
           lin0_W, lin0_b, lin1_W, lin1_b, wprop_W, wprop_b):
    f32 = jnp.float32
    src = edge_index[0].astype(jnp.int32)
    dst = edge_index[1].astype(jnp.int32)
    x2 = x.astype(jnp.int32).reshape(N, 1)
    batch2 = batch.astype(jnp.int32).reshape(N, 1)

    w1_0 = (conv0_W1 * _EMB_SCALE).astype(f32)
    w1_1 = (conv1_W1 * _EMB_SCALE).astype(f32)
    w2_0 = (conv0_W2 * (np.sqrt(2.0) / np.sqrt(FC_HIDDEN))).astype(f32)
    w2_1 = (conv1_W2 * (np.sqrt(2.0) / np.sqrt(FC_HIDDEN))).astype(f32)
    rep = jnp.asarray(_REP)
    sel = jnp.asarray(_SEL * _OUT_SCALE)
    zeros = jnp.zeros((N, H), f32)

    h0 = _embed_call(x2, embed.astype(f32))
    x1 = _make_gather(1)(h0, src)
    if isinstance(x1, (list, tuple)):
        x1 = x1[0]
    ef0 = _edge_call1(edge_attr, x1, w1_0, w2_0, rep, sel)
    p0a, p0b = _make_scatter()(ef0, dst, zeros)
    x1a, x1b = _make_gather(2)(p0a, p0b, src)
    ef1 = _edge_call2(edge_attr, x1a, x1b, w1_1, w2_1, rep, sel)
    p1a, p1b = _make_scatter()(ef1, dst, zeros)

    return _tail_call(p1a, p1b, batch2,
                      lin0_W.astype(f32), lin0_b.reshape(1, H).astype(f32),
                      lin1_W.astype(f32), lin1_b.reshape(1, H).astype(f32),
                      wprop_W.astype(f32), wprop_b.reshape(1, 1).astype(f32))
